# baseline (speedup 1.0000x reference)
import functools

import jax
import jax.numpy as jnp
from jax import lax
from jax.experimental import pallas as pl
from jax.experimental.pallas import tpu as pltpu
from jax.experimental.pallas import tpu_sc as plsc

FIELDS = 26
DIM = 32
BATCH = 16384
V = FIELDS * 100000
NC, NS, L = 2, 16, 16
NW = NC * NS
SUB = 128
GROUP = 4
UNITS = FIELDS * (BATCH // (SUB * GROUP))
UNITS_W = UNITS // NW
BBLKS = BATCH // (SUB * GROUP)

_mesh = plsc.VectorSubcoreMesh(core_axis_name="c", subcore_axis_name="s")


TBLK = 512
TGRID = (V + TBLK - 1) // TBLK


def _transpose_block(tin_ref, tout_ref):
    x = tin_ref[...]
    e = (lax.broadcasted_iota(jnp.int32, (DIM, DIM), 0)
         == lax.broadcasted_iota(jnp.int32, (DIM, DIM), 1)).astype(jnp.float32)
    tout_ref[...] = lax.dot_general(x, e, (((0,), (0,)), ((), ())),
                                    precision=lax.Precision.HIGHEST,
                                    preferred_element_type=jnp.float32)


_table_transpose = pl.pallas_call(
    _transpose_block,
    grid=(TGRID,),
    in_specs=[pl.BlockSpec((DIM, TBLK), lambda j: (0, j))],
    out_specs=pl.BlockSpec((TBLK, DIM), lambda j: (j, 0)),
    out_shape=jax.ShapeDtypeStruct((V, DIM), jnp.float32),
)


@functools.partial(
    pl.kernel,
    out_type=jax.ShapeDtypeStruct((FIELDS, BATCH // SUB, SUB, DIM),
                                  jnp.float32),
    mesh=_mesh,
    compiler_params=pltpu.CompilerParams(use_tc_tiling_on_sc=False),
    scratch_types=[
        pltpu.VMEM((3 * L,), jnp.int32),
        pltpu.VMEM((GROUP, SUB), jnp.int32),
        pltpu.VMEM((GROUP, SUB, DIM), jnp.float32),
        pltpu.SemaphoreType.DMA,
    ],
)
def _buckle_gather(idx_hbm, off_hbm, table_hbm, out_hbm,
                   off_v, idx_v, rows_v, sem):
    wid = lax.axis_index("s") * NC + lax.axis_index("c")
    pltpu.sync_copy(off_hbm, off_v)

    def unit_body(c, carry):
        u = wid * UNITS_W + c
        f = u // BBLKS
        jb = (u % BBLKS) * GROUP
        foff = off_v[pl.ds(f, L)][0]
        pltpu.sync_copy(idx_hbm.at[f, pl.ds(jb, GROUP)], idx_v)
        for j in range(GROUP):
            for s in range(SUB // L):
                sl = pl.ds(s * L, L)
                idx_v[j, sl] = idx_v[j, sl] + foff
        copies = [
            pltpu.async_copy(table_hbm.at[idx_v.at[j]], rows_v.at[j], sem)
            for j in range(GROUP)
        ]
        for cp in copies:
            cp.wait()
        pltpu.sync_copy(rows_v, out_hbm.at[f, pl.ds(jb, GROUP)])
        return carry

    lax.fori_loop(0, UNITS_W, unit_body, 0)


def kernel(categorical_inputs, embedding_weight, offsets):
    table_rm = _table_transpose(embedding_weight.T)
    idx_fm = (categorical_inputs.astype(jnp.int32).T
              .reshape(FIELDS, BATCH // SUB, SUB))
    off = jnp.pad(offsets[:FIELDS].astype(jnp.int32), (0, 3 * L - FIELDS))
    out2 = _buckle_gather(idx_fm, off, table_rm)
    return jnp.transpose(out2.reshape(FIELDS, BATCH, DIM), (1, 0, 2))


# speedup vs baseline: 2.6043x; 2.6043x over previous
import functools

import jax
import jax.numpy as jnp
from jax import lax
from jax.experimental import pallas as pl
from jax.experimental.pallas import tpu as pltpu
from jax.experimental.pallas import tpu_sc as plsc

FIELDS = 26
DIM = 32
BATCH = 16384
V = FIELDS * 100000
NC, NS, L = 2, 16, 16
NW = NC * NS
SUB = 128
GROUP = 4
UNITS = FIELDS * (BATCH // (SUB * GROUP))
UNITS_W = UNITS // NW
BBLKS = BATCH // (SUB * GROUP)

_mesh = plsc.VectorSubcoreMesh(core_axis_name="c", subcore_axis_name="s")


TBLK = 8192
TGRID = (V + TBLK - 1) // TBLK


def _transpose_block(tin_ref, tout_ref):
    tout_ref[...] = tin_ref[...].T


_table_transpose = pl.pallas_call(
    _transpose_block,
    grid=(TGRID,),
    in_specs=[pl.BlockSpec((DIM, TBLK), lambda j: (0, j))],
    out_specs=pl.BlockSpec((TBLK, DIM), lambda j: (j, 0)),
    out_shape=jax.ShapeDtypeStruct((V, DIM), jnp.float32),
)


KBLK = 8192


def _out_transpose_block(oin_ref, oout_ref):
    oout_ref[...] = jnp.transpose(oin_ref[...], (0, 2, 1))


_out_transpose = pl.pallas_call(
    _out_transpose_block,
    grid=(FIELDS, BATCH // KBLK),
    in_specs=[pl.BlockSpec((1, KBLK, DIM), lambda f, j: (f, j, 0))],
    out_specs=pl.BlockSpec((1, DIM, KBLK), lambda f, j: (f, 0, j)),
    out_shape=jax.ShapeDtypeStruct((FIELDS, DIM, BATCH), jnp.float32),
)


@functools.partial(
    pl.kernel,
    out_type=jax.ShapeDtypeStruct((FIELDS, BATCH // SUB, SUB, DIM),
                                  jnp.float32),
    mesh=_mesh,
    compiler_params=pltpu.CompilerParams(use_tc_tiling_on_sc=False),
    scratch_types=[
        pltpu.VMEM((3 * L,), jnp.int32),
        pltpu.VMEM((GROUP, SUB), jnp.int32),
        pltpu.VMEM((GROUP, SUB, DIM), jnp.float32),
        pltpu.SemaphoreType.DMA,
    ],
)
def _buckle_gather(idx_hbm, off_hbm, table_hbm, out_hbm,
                   off_v, idx_v, rows_v, sem):
    wid = lax.axis_index("s") * NC + lax.axis_index("c")
    pltpu.sync_copy(off_hbm, off_v)

    def unit_body(c, carry):
        u = wid * UNITS_W + c
        f = u // BBLKS
        jb = (u % BBLKS) * GROUP
        foff = off_v[pl.ds(f, L)][0]
        pltpu.sync_copy(idx_hbm.at[f, pl.ds(jb, GROUP)], idx_v)
        for j in range(GROUP):
            for s in range(SUB // L):
                sl = pl.ds(s * L, L)
                idx_v[j, sl] = idx_v[j, sl] + foff
        copies = [
            pltpu.async_copy(table_hbm.at[idx_v.at[j]], rows_v.at[j], sem)
            for j in range(GROUP)
        ]
        for cp in copies:
            cp.wait()
        pltpu.sync_copy(rows_v, out_hbm.at[f, pl.ds(jb, GROUP)])
        return carry

    lax.fori_loop(0, UNITS_W, unit_body, 0)


def kernel(categorical_inputs, embedding_weight, offsets):
    table_rm = _table_transpose(embedding_weight.T)
    idx_fm = (categorical_inputs.astype(jnp.int32).T
              .reshape(FIELDS, BATCH // SUB, SUB))
    off = jnp.pad(offsets[:FIELDS].astype(jnp.int32), (0, 3 * L - FIELDS))
    out2 = _buckle_gather(idx_fm, off, table_rm)
    outT = _out_transpose(out2.reshape(FIELDS, BATCH, DIM))
    return jnp.transpose(outT, (2, 0, 1))


# speedup vs baseline: 4.6845x; 1.7988x over previous
import functools

import jax
import jax.numpy as jnp
from jax import lax
from jax.experimental import pallas as pl
from jax.experimental.pallas import tpu as pltpu
from jax.experimental.pallas import tpu_sc as plsc

FIELDS = 26
DIM = 32
BATCH = 16384
V = FIELDS * 100000
NC, NS, L = 2, 16, 16
NW = NC * NS
SUB = 128
GROUP = 4
UNITS = FIELDS * (BATCH // (SUB * GROUP))
UNITS_W = UNITS // NW
BBLKS = BATCH // (SUB * GROUP)

_mesh = plsc.VectorSubcoreMesh(core_axis_name="c", subcore_axis_name="s")


TBLK = 8192
TGRID = (V + TBLK - 1) // TBLK


def _transpose_block(tin_ref, tout_ref):
    xt = tin_ref[...].T
    tout_ref[...] = jnp.concatenate(
        [xt, jnp.zeros((TBLK, 128 - DIM), jnp.float32)], axis=1)


_table_transpose = pl.pallas_call(
    _transpose_block,
    grid=(TGRID,),
    in_specs=[pl.BlockSpec((DIM, TBLK), lambda j: (0, j))],
    out_specs=pl.BlockSpec((TBLK, 128), lambda j: (j, 0)),
    out_shape=jax.ShapeDtypeStruct((V, 128), jnp.float32),
)


@functools.partial(
    pl.kernel,
    out_type=jax.ShapeDtypeStruct((FIELDS, BATCH, DIM), jnp.float32),
    mesh=_mesh,
    compiler_params=pltpu.CompilerParams(use_tc_tiling_on_sc=False),
    scratch_types=[
        pltpu.VMEM((3 * L,), jnp.int32),
        pltpu.VMEM((GROUP, SUB), jnp.int32),
        pltpu.VMEM((GROUP * SUB, 128), jnp.float32),
        pltpu.SemaphoreType.DMA,
    ],
)
def _buckle_gather(idx_hbm, off_hbm, table_hbm, out_hbm,
                   off_v, idx_v, rows_v, sem):
    wid = lax.axis_index("s") * NC + lax.axis_index("c")
    pltpu.sync_copy(off_hbm, off_v)

    def unit_body(c, carry):
        u = wid * UNITS_W + c
        f = u // BBLKS
        jb = (u % BBLKS) * GROUP
        foff = off_v[pl.ds(f, L)][0]
        pltpu.sync_copy(idx_hbm.at[f, pl.ds(jb, GROUP)], idx_v)
        for j in range(GROUP):
            for s in range(SUB // L):
                sl = pl.ds(s * L, L)
                idx_v[j, sl] = idx_v[j, sl] + foff
        copies = [
            pltpu.async_copy(table_hbm.at[idx_v.at[j]],
                             rows_v.at[pl.ds(j * SUB, SUB)], sem)
            for j in range(GROUP)
        ]
        for cp in copies:
            cp.wait()
        pltpu.sync_copy(rows_v.at[pl.ds(0, GROUP * SUB), pl.ds(0, DIM)],
                        out_hbm.at[f, pl.ds(jb * SUB, GROUP * SUB)])
        return carry

    lax.fori_loop(0, UNITS_W, unit_body, 0)


def kernel(categorical_inputs, embedding_weight, offsets):
    table_rm = _table_transpose(embedding_weight.T)
    idx_fm = (categorical_inputs.astype(jnp.int32).T
              .reshape(FIELDS, BATCH // SUB, SUB))
    off = jnp.pad(offsets[:FIELDS].astype(jnp.int32), (0, 3 * L - FIELDS))
    out2 = _buckle_gather(idx_fm, off, table_rm)
    return jnp.transpose(out2, (1, 0, 2))


# speedup vs baseline: 5.2446x; 1.1196x over previous
import functools

import jax
import jax.numpy as jnp
from jax import lax
from jax.experimental import pallas as pl
from jax.experimental.pallas import tpu as pltpu
from jax.experimental.pallas import tpu_sc as plsc

FIELDS = 26
DIM = 32
BATCH = 16384
V = FIELDS * 100000
NC, NS, L = 2, 16, 16
NW = NC * NS
SUB = 128
GROUP = 4
UNITS = FIELDS * (BATCH // (SUB * GROUP))
UNITS_W = UNITS // NW
BBLKS = BATCH // (SUB * GROUP)

_mesh = plsc.VectorSubcoreMesh(core_axis_name="c", subcore_axis_name="s")


TBLK = 8192
TGRID = (V + TBLK - 1) // TBLK


def _transpose_block(tin_ref, tout_ref):
    xt = tin_ref[...].T
    tout_ref[...] = jnp.concatenate(
        [xt, jnp.zeros((TBLK, 128 - DIM), jnp.float32)], axis=1)


_table_transpose = pl.pallas_call(
    _transpose_block,
    grid=(TGRID,),
    in_specs=[pl.BlockSpec((DIM, TBLK), lambda j: (0, j))],
    out_specs=pl.BlockSpec((TBLK, 128), lambda j: (j, 0)),
    out_shape=jax.ShapeDtypeStruct((V, 128), jnp.float32),
)


@functools.partial(
    pl.kernel,
    out_type=jax.ShapeDtypeStruct((FIELDS, BATCH, DIM), jnp.float32),
    mesh=_mesh,
    compiler_params=pltpu.CompilerParams(use_tc_tiling_on_sc=False),
    scratch_types=[
        pltpu.VMEM((3 * L,), jnp.int32),
        pltpu.VMEM((GROUP, SUB), jnp.int32),
        pltpu.VMEM((GROUP * SUB, DIM), jnp.float32),
        pltpu.SemaphoreType.DMA,
    ],
)
def _buckle_gather(idx_hbm, off_hbm, table_hbm, out_hbm,
                   off_v, idx_v, rows_v, sem):
    wid = lax.axis_index("s") * NC + lax.axis_index("c")
    pltpu.sync_copy(off_hbm, off_v)

    def unit_body(c, carry):
        u = wid * UNITS_W + c
        f = u // BBLKS
        jb = (u % BBLKS) * GROUP
        foff = off_v[pl.ds(f, L)][0]
        pltpu.sync_copy(idx_hbm.at[f, pl.ds(jb, GROUP)], idx_v)
        for j in range(GROUP):
            for s in range(SUB // L):
                sl = pl.ds(s * L, L)
                idx_v[j, sl] = (idx_v[j, sl] + foff) * 4
        copies = [
            pltpu.async_copy(table_hbm.at[idx_v.at[j]],
                             rows_v.at[pl.ds(j * SUB, SUB)], sem)
            for j in range(GROUP)
        ]
        for cp in copies:
            cp.wait()
        pltpu.sync_copy(rows_v, out_hbm.at[f, pl.ds(jb * SUB, GROUP * SUB)])
        return carry

    lax.fori_loop(0, UNITS_W, unit_body, 0)


def kernel(categorical_inputs, embedding_weight, offsets):
    table_rm = _table_transpose(embedding_weight.T)
    idx_fm = (categorical_inputs.astype(jnp.int32).T
              .reshape(FIELDS, BATCH // SUB, SUB))
    off = jnp.pad(offsets[:FIELDS].astype(jnp.int32), (0, 3 * L - FIELDS))
    out2 = _buckle_gather(idx_fm, off, table_rm.reshape(4 * V, DIM))
    return jnp.transpose(out2, (1, 0, 2))
